# baseline (speedup 1.0000x reference)
import functools

import jax
import jax.numpy as jnp
from jax import lax
from jax.experimental import pallas as pl
from jax.experimental.pallas import tpu as pltpu
from jax.experimental.pallas import tpu_sc as plsc

_D = 64
_DW = _D // 2
_EXP_BUMP = 0x01800180
_NC = 2
_NS = 16
_NW = _NC * _NS
_CHUNK = 128


@functools.lru_cache(maxsize=None)
def _build(total: int, per_w: int, nchunk: int):
    mesh = plsc.VectorSubcoreMesh(core_axis_name="c", subcore_axis_name="s")

    @functools.partial(
        pl.kernel,
        mesh=mesh,
        compiler_params=pltpu.CompilerParams(use_tc_tiling_on_sc=False),
        out_type=jax.ShapeDtypeStruct((total, _DW), jnp.int32),
        scratch_types=[
            pltpu.VMEM((nchunk, _CHUNK), jnp.int32),
            pltpu.VMEM((_CHUNK, _DW), jnp.int32),
            pltpu.SemaphoreType.DMA,
        ],
    )
    def k(idx_hbm, tab_hbm, out_hbm, idx_v, rows_v, sem):
        wid = lax.axis_index("s") * _NC + lax.axis_index("c")
        base = wid * per_w
        pltpu.sync_copy(idx_hbm.at[wid], idx_v)

        def chunk_body(j, carry):
            pltpu.async_copy(tab_hbm.at[idx_v.at[j]], rows_v, sem).wait()

            def row_body(i, c):
                for cc in range(2):
                    sl = (i, pl.ds(16 * cc, 16))
                    rows_v[sl] = rows_v[sl] + jnp.int32(_EXP_BUMP)
                return c

            lax.fori_loop(0, _CHUNK, row_body, 0, unroll=4)
            pltpu.sync_copy(
                rows_v, out_hbm.at[pl.ds(base + j * _CHUNK, _CHUNK)]
            )
            return carry

        lax.fori_loop(0, nchunk, chunk_body, 0)

    return k


def kernel(x, table):
    b, h = x.shape
    n, d = table.shape
    total = b * h
    per_w = total // _NW
    nchunk = per_w // _CHUNK
    assert per_w * _NW == total and nchunk * _CHUNK == per_w and d == _D
    x_flat = x.reshape(_NW, per_w // _CHUNK, _CHUNK)
    tab_i32 = jax.lax.bitcast_convert_type(
        table.reshape(n, _DW, 2), jnp.int32
    )
    out = _build(total, per_w, nchunk)(x_flat, tab_i32)
    out_bf16 = jax.lax.bitcast_convert_type(out, jnp.bfloat16)
    return out_bf16.reshape(b, h, _D)


# speedup vs baseline: 1.0106x; 1.0106x over previous
import functools

import jax
import jax.numpy as jnp
from jax import lax
from jax.experimental import pallas as pl
from jax.experimental.pallas import tpu as pltpu
from jax.experimental.pallas import tpu_sc as plsc

_D = 64
_DW = _D // 2
_EXP_BUMP = 0x01800180
_NC = 2
_NS = 16
_NW = _NC * _NS
_CHUNK = 128
_NBUF = 4


@functools.lru_cache(maxsize=None)
def _build(total: int, per_w: int, nchunk: int):
    mesh = plsc.VectorSubcoreMesh(core_axis_name="c", subcore_axis_name="s")
    nouter = nchunk // _NBUF

    @functools.partial(
        pl.kernel,
        mesh=mesh,
        compiler_params=pltpu.CompilerParams(use_tc_tiling_on_sc=False),
        out_type=jax.ShapeDtypeStruct((total, _DW), jnp.int32),
        scratch_types=[
            pltpu.VMEM((nchunk, _CHUNK), jnp.int32),
            pltpu.VMEM((_NBUF, _CHUNK, _DW), jnp.int32),
            pltpu.VMEM((_NBUF, _CHUNK, _DW), jnp.int32),
        ]
        + [pltpu.SemaphoreType.DMA] * (2 * _NBUF),
    )
    def k(idx_hbm, tab_hbm, out_hbm, idx_v, rin, rout, *sems):
        gsem = sems[:_NBUF]
        ssem = sems[_NBUF:]
        wid = lax.axis_index("s") * _NC + lax.axis_index("c")
        base = wid * per_w
        pltpu.sync_copy(idx_hbm.at[wid], idx_v)

        def gissue(c, b):
            pltpu.async_copy(tab_hbm.at[idx_v.at[c]], rin.at[b], gsem[b])

        def gwait(c, b):
            pltpu.make_async_copy(
                tab_hbm.at[idx_v.at[c]], rin.at[b], gsem[b]
            ).wait()

        def sissue(c, b):
            pltpu.async_copy(
                rout.at[b], out_hbm.at[pl.ds(base + c * _CHUNK, _CHUNK)],
                ssem[b],
            )

        def swait(c, b):
            pltpu.make_async_copy(
                rout.at[b], out_hbm.at[pl.ds(base + c * _CHUNK, _CHUNK)],
                ssem[b],
            ).wait()

        def scale(b):
            def body(i, carry):
                for cc in range(2):
                    sl = pl.ds(16 * cc, 16)
                    rout[b, i, sl] = rin[b, i, sl] + jnp.int32(_EXP_BUMP)
                return carry

            lax.fori_loop(0, _CHUNK, body, 0, unroll=8)

        for b in range(_NBUF):
            gissue(b, b)
        for b in range(_NBUF):
            gwait(b, b)
            scale(b)
            sissue(b, b)
            gissue(b + _NBUF, b)

        def outer(j, carry):
            for b in range(_NBUF):
                c = j * _NBUF + b
                gwait(c, b)
                scale(b)
                swait(c - _NBUF, b)
                sissue(c, b)
                gissue(c + _NBUF, b)
            return carry

        lax.fori_loop(1, nouter - 1, outer, 0)

        for b in range(_NBUF):
            c = (nouter - 1) * _NBUF + b
            gwait(c, b)
            scale(b)
            swait(c - _NBUF, b)
            sissue(c, b)
        for b in range(_NBUF):
            swait((nouter - 1) * _NBUF + b, b)

    return k


def kernel(x, table):
    b, h = x.shape
    n, d = table.shape
    total = b * h
    per_w = total // _NW
    nchunk = per_w // _CHUNK
    assert per_w * _NW == total and nchunk * _CHUNK == per_w and d == _D
    assert nchunk % _NBUF == 0 and nchunk // _NBUF >= 2
    x_resh = x.reshape(_NW, nchunk, _CHUNK)
    tab_i32 = jax.lax.bitcast_convert_type(
        table.reshape(n, _DW, 2), jnp.int32
    )
    out = _build(total, per_w, nchunk)(x_resh, tab_i32)
    out_bf16 = jax.lax.bitcast_convert_type(out, jnp.bfloat16)
    return out_bf16.reshape(b, h, _D)


# speedup vs baseline: 1.6297x; 1.6125x over previous
import functools

import jax
import jax.numpy as jnp
from jax import lax
from jax.experimental import pallas as pl
from jax.experimental.pallas import tpu as pltpu
from jax.experimental.pallas import tpu_sc as plsc

_D = 64
_DW = _D // 2
_EXP_BUMP = 0x01800180
_NC = 2
_NS = 16
_NW = _NC * _NS
_CHUNK = 128
_NBUF = 4
_SEG = 512


@functools.lru_cache(maxsize=None)
def _build(bsz: int, hist: int, nchunk: int):
    mesh = plsc.VectorSubcoreMesh(core_axis_name="c", subcore_axis_name="s")
    nouter = nchunk // _NBUF
    blocks_per_h = bsz // _CHUNK
    out_rows = hist * 8 * blocks_per_h * _SEG // 1024

    @functools.partial(
        pl.kernel,
        mesh=mesh,
        compiler_params=pltpu.CompilerParams(
            use_tc_tiling_on_sc=False, needs_layout_passes=False
        ),
        out_type=jax.ShapeDtypeStruct((out_rows, 1024), jnp.int32),
        scratch_types=[
            pltpu.VMEM((nchunk, _CHUNK), jnp.int32),
            pltpu.VMEM((_NBUF, _CHUNK, _D), jnp.float32),
            pltpu.VMEM((_NBUF, _DW * _CHUNK), jnp.int32),
        ]
        + [pltpu.SemaphoreType.DMA] * (2 * _NBUF),
    )
    def k(idx_hbm, tab_hbm, out_hbm, idx_v, g, st, *sems):
        gsem = sems[:_NBUF]
        ssem = sems[_NBUF:]
        wid = lax.axis_index("s") * _NC + lax.axis_index("c")
        pltpu.sync_copy(idx_hbm.at[wid], idx_v)
        iota = lax.iota(jnp.int32, 16)
        mask_hi = jnp.full((16,), -0x10000, dtype=jnp.int32)

        def gissue(c, b):
            pltpu.async_copy(tab_hbm.at[idx_v.at[c]], g.at[b], gsem[b])

        def gwait(c, b):
            pltpu.make_async_copy(
                tab_hbm.at[idx_v.at[c]], g.at[b], gsem[b]
            ).wait()

        def pack(b):
            gb = g.at[b]
            zeros = iota * 0

            def body(p, carry):
                for lg in range(8):
                    rows = iota + 16 * lg
                    ev = plsc.bitcast(
                        plsc.load_gather(gb, [rows, zeros + 2 * p]),
                        jnp.int32,
                    )
                    od = plsc.bitcast(
                        plsc.load_gather(gb, [rows, zeros + 2 * p + 1]),
                        jnp.int32,
                    )
                    w = lax.shift_right_logical(ev, 16) | (od & mask_hi)
                    st[b, pl.ds(p * _CHUNK + 16 * lg, 16)] = (
                        w + jnp.int32(_EXP_BUMP)
                    )
                return carry

            lax.fori_loop(0, _DW, body, 0)

        def out_word_base(c, e_tile):
            gchunk = wid * nchunk + c
            h = gchunk // blocks_per_h
            bt = lax.rem(gchunk, blocks_per_h)
            return ((h * 8 + e_tile) * blocks_per_h + bt) * _SEG

        def sissue(c, b):
            for e_tile in range(8):
                w = out_word_base(c, e_tile)
                pltpu.async_copy(
                    st.at[b, pl.ds(e_tile * _SEG, _SEG)],
                    out_hbm.at[w // 1024, pl.ds(lax.rem(w, 1024), _SEG)],
                    ssem[b],
                )

        def swait(c, b):
            for e_tile in range(8):
                w = out_word_base(c, e_tile)
                pltpu.make_async_copy(
                    st.at[b, pl.ds(e_tile * _SEG, _SEG)],
                    out_hbm.at[w // 1024, pl.ds(lax.rem(w, 1024), _SEG)],
                    ssem[b],
                ).wait()

        for b in range(_NBUF):
            gissue(b, b)
        for b in range(_NBUF):
            gwait(b, b)
            pack(b)
            sissue(b, b)
            gissue(b + _NBUF, b)

        def outer(j, carry):
            for b in range(_NBUF):
                c = j * _NBUF + b
                gwait(c, b)
                swait(c - _NBUF, b)
                pack(b)
                sissue(c, b)
                gissue(c + _NBUF, b)
            return carry

        lax.fori_loop(1, nouter - 1, outer, 0)

        for b in range(_NBUF):
            c = (nouter - 1) * _NBUF + b
            gwait(c, b)
            swait(c - _NBUF, b)
            pack(b)
            sissue(c, b)
        for b in range(_NBUF):
            swait((nouter - 1) * _NBUF + b, b)

    return k


def kernel(x, table):
    b, h = x.shape
    n, d = table.shape
    total = b * h
    per_w = total // _NW
    nchunk = per_w // _CHUNK
    assert per_w * _NW == total and nchunk * _CHUNK == per_w and d == _D
    assert nchunk % _NBUF == 0 and nchunk // _NBUF >= 2
    assert b % (2 * _CHUNK) == 0
    xt = x.T.reshape(_NW, nchunk, _CHUNK)
    tab_f32 = table.astype(jnp.float32)
    out = _build(b, h, nchunk)(xt, tab_f32)
    out_bf = jax.lax.bitcast_convert_type(out, jnp.bfloat16)
    out6 = out_bf.reshape(h, 8, b // _CHUNK, 4, _CHUNK, 2)
    return out6.transpose(2, 4, 0, 1, 3, 5).reshape(b, h, _D)


# speedup vs baseline: 1.7827x; 1.0939x over previous
import functools

import jax
import jax.numpy as jnp
from jax import lax
from jax.experimental import pallas as pl
from jax.experimental.pallas import tpu as pltpu
from jax.experimental.pallas import tpu_sc as plsc

_D = 64
_DW = _D // 2
_EXP_BUMP = 0x01800180
_NC = 2
_NS = 16
_NW = _NC * _NS
_CHUNK = 128
_NBUF = 4
_SEG = 512


@functools.lru_cache(maxsize=None)
def _build(bsz: int, hist: int, nchunk: int):
    mesh = plsc.VectorSubcoreMesh(core_axis_name="c", subcore_axis_name="s")
    nouter = nchunk // _NBUF
    blocks_per_h = bsz // _CHUNK
    out_rows = hist * 8 * blocks_per_h * _SEG // 1024

    @functools.partial(
        pl.kernel,
        mesh=mesh,
        compiler_params=pltpu.CompilerParams(
            use_tc_tiling_on_sc=False, needs_layout_passes=False
        ),
        out_type=jax.ShapeDtypeStruct((out_rows * 1024,), jnp.int32),
        scratch_types=[
            pltpu.VMEM((nchunk, _CHUNK), jnp.int32),
            pltpu.VMEM((_NBUF, _CHUNK, _D), jnp.float32),
            pltpu.VMEM((_NBUF, _DW * _CHUNK), jnp.int32),
        ]
        + [pltpu.SemaphoreType.DMA] * (2 * _NBUF),
    )
    def k(idx_hbm, tab_hbm, out_hbm, idx_v, g, st, *sems):
        gsem = sems[:_NBUF]
        ssem = sems[_NBUF:]
        wid = lax.axis_index("s") * _NC + lax.axis_index("c")
        pltpu.sync_copy(idx_hbm.at[wid], idx_v)
        iota = lax.iota(jnp.int32, 16)
        mask_hi = jnp.full((16,), -0x10000, dtype=jnp.int32)

        def gissue(c, b):
            pltpu.async_copy(tab_hbm.at[idx_v.at[c]], g.at[b], gsem[b])

        def gwait(c, b):
            pltpu.make_async_copy(
                tab_hbm.at[idx_v.at[c]], g.at[b], gsem[b]
            ).wait()

        def pack(b):
            gb = g.at[b]
            zeros = iota * 0
            rows = [iota + 16 * lg for lg in range(8)]

            def body(p, carry):
                cols0 = zeros + 2 * p
                cols1 = cols0 + 1
                base = p * _CHUNK
                for lg in range(8):
                    ev = plsc.bitcast(
                        plsc.load_gather(gb, [rows[lg], cols0]), jnp.int32
                    )
                    od = plsc.bitcast(
                        plsc.load_gather(gb, [rows[lg], cols1]), jnp.int32
                    )
                    w = lax.shift_right_logical(ev, 16) | (od & mask_hi)
                    st[b, pl.ds(base + 16 * lg, 16)] = (
                        w + jnp.int32(_EXP_BUMP)
                    )
                return carry

            lax.fori_loop(0, _DW, body, 0, unroll=4)

        def out_word_base(c, e_tile):
            gchunk = wid * nchunk + c
            h = gchunk // blocks_per_h
            bt = lax.rem(gchunk, blocks_per_h)
            return ((h * 8 + e_tile) * blocks_per_h + bt) * _SEG

        def sissue(c, b):
            for e_tile in range(8):
                w = out_word_base(c, e_tile)
                pltpu.async_copy(
                    st.at[b, pl.ds(e_tile * _SEG, _SEG)],
                    out_hbm.at[pl.ds(w, _SEG)],
                    ssem[b],
                )

        def swait(c, b):
            for e_tile in range(8):
                w = out_word_base(c, e_tile)
                pltpu.make_async_copy(
                    st.at[b, pl.ds(e_tile * _SEG, _SEG)],
                    out_hbm.at[pl.ds(w, _SEG)],
                    ssem[b],
                ).wait()

        for b in range(_NBUF):
            gissue(b, b)
        for b in range(_NBUF):
            gwait(b, b)
            pack(b)
            sissue(b, b)
            gissue(b + _NBUF, b)

        def outer(j, carry):
            for b in range(_NBUF):
                c = j * _NBUF + b
                gwait(c, b)
                swait(c - _NBUF, b)
                pack(b)
                sissue(c, b)
                gissue(c + _NBUF, b)
            return carry

        lax.fori_loop(1, nouter - 1, outer, 0)

        for b in range(_NBUF):
            c = (nouter - 1) * _NBUF + b
            gwait(c, b)
            swait(c - _NBUF, b)
            pack(b)
            sissue(c, b)
        for b in range(_NBUF):
            swait((nouter - 1) * _NBUF + b, b)

    return k


def kernel(x, table):
    b, h = x.shape
    n, d = table.shape
    total = b * h
    per_w = total // _NW
    nchunk = per_w // _CHUNK
    assert per_w * _NW == total and nchunk * _CHUNK == per_w and d == _D
    assert nchunk % _NBUF == 0 and nchunk // _NBUF >= 2
    assert b % (2 * _CHUNK) == 0
    xt = x.T.reshape(_NW, nchunk, _CHUNK)
    tab_f32 = table.astype(jnp.float32)
    out = _build(b, h, nchunk)(xt, tab_f32)
    out_bf = jax.lax.bitcast_convert_type(out, jnp.bfloat16)
    out6 = out_bf.reshape(h, 8, b // _CHUNK, 4, _CHUNK, 2)
    return out6.transpose(2, 4, 0, 1, 3, 5).reshape(b, h, _D)


# speedup vs baseline: 2.1297x; 1.1946x over previous
import functools

import jax
import jax.numpy as jnp
from jax import lax
from jax.experimental import pallas as pl
from jax.experimental.pallas import tpu as pltpu
from jax.experimental.pallas import tpu_sc as plsc

_D = 64
_DW = _D // 2
_EXP_BUMP = 0x01800180
_NC = 2
_NS = 16
_NW = _NC * _NS
_CHUNK = 128
_NBUF = 4
_SEG = 512


@functools.lru_cache(maxsize=None)
def _build(bsz: int, hist: int, nchunk: int):
    mesh = plsc.VectorSubcoreMesh(core_axis_name="c", subcore_axis_name="s")
    nouter = nchunk // _NBUF
    blocks_per_h = bsz // _CHUNK
    out_rows = hist * 8 * blocks_per_h * _SEG // 1024

    @functools.partial(
        pl.kernel,
        mesh=mesh,
        compiler_params=pltpu.CompilerParams(
            use_tc_tiling_on_sc=False, needs_layout_passes=False
        ),
        out_type=jax.ShapeDtypeStruct((out_rows * 1024,), jnp.int32),
        scratch_types=[
            pltpu.VMEM((nchunk, _CHUNK), jnp.int32),
            pltpu.VMEM((_NBUF, _CHUNK, _D), jnp.float32),
            pltpu.VMEM((_NBUF, _DW * _CHUNK), jnp.int32),
        ]
        + [pltpu.SemaphoreType.DMA] * (2 * _NBUF),
    )
    def k(idx_hbm, tab_hbm, out_hbm, idx_v, g, st, *sems):
        gsem = sems[:_NBUF]
        ssem = sems[_NBUF:]
        wid = lax.axis_index("s") * _NC + lax.axis_index("c")
        pltpu.sync_copy(idx_hbm.at[wid], idx_v)
        iota = lax.iota(jnp.int32, 16)
        mask_hi = jnp.full((16,), -0x10000, dtype=jnp.int32)

        def gissue(c, b):
            pltpu.async_copy(tab_hbm.at[idx_v.at[c]], g.at[b], gsem[b])

        def gwait(c, b):
            pltpu.make_async_copy(
                tab_hbm.at[idx_v.at[c]], g.at[b], gsem[b]
            ).wait()

        def pack(b):
            gb = g.at[b]
            zeros = iota * 0
            rows = [iota + 16 * lg for lg in range(8)]

            @plsc.parallel_loop(0, _DW, unroll=2)
            def body(p):
                cols0 = zeros + 2 * p
                cols1 = cols0 + 1
                base = p * _CHUNK
                for lg in range(8):
                    ev = plsc.bitcast(
                        plsc.load_gather(gb, [rows[lg], cols0]), jnp.int32
                    )
                    od = plsc.bitcast(
                        plsc.load_gather(gb, [rows[lg], cols1]), jnp.int32
                    )
                    w = lax.shift_right_logical(ev, 16) | (od & mask_hi)
                    st[b, pl.ds(base + 16 * lg, 16)] = (
                        w + jnp.int32(_EXP_BUMP)
                    )

        def out_word_base(c, e_tile):
            gchunk = wid * nchunk + c
            h = gchunk // blocks_per_h
            bt = lax.rem(gchunk, blocks_per_h)
            return ((h * 8 + e_tile) * blocks_per_h + bt) * _SEG

        def sissue(c, b):
            for e_tile in range(8):
                w = out_word_base(c, e_tile)
                pltpu.async_copy(
                    st.at[b, pl.ds(e_tile * _SEG, _SEG)],
                    out_hbm.at[pl.ds(w, _SEG)],
                    ssem[b],
                )

        def swait(c, b):
            for e_tile in range(8):
                w = out_word_base(c, e_tile)
                pltpu.make_async_copy(
                    st.at[b, pl.ds(e_tile * _SEG, _SEG)],
                    out_hbm.at[pl.ds(w, _SEG)],
                    ssem[b],
                ).wait()

        for b in range(_NBUF):
            gissue(b, b)
        for b in range(_NBUF):
            gwait(b, b)
            pack(b)
            sissue(b, b)
            gissue(b + _NBUF, b)

        def outer(j, carry):
            for b in range(_NBUF):
                c = j * _NBUF + b
                gwait(c, b)
                swait(c - _NBUF, b)
                pack(b)
                sissue(c, b)
                gissue(c + _NBUF, b)
            return carry

        lax.fori_loop(1, nouter - 1, outer, 0)

        for b in range(_NBUF):
            c = (nouter - 1) * _NBUF + b
            gwait(c, b)
            swait(c - _NBUF, b)
            pack(b)
            sissue(c, b)
        for b in range(_NBUF):
            swait((nouter - 1) * _NBUF + b, b)

    return k


def kernel(x, table):
    b, h = x.shape
    n, d = table.shape
    total = b * h
    per_w = total // _NW
    nchunk = per_w // _CHUNK
    assert per_w * _NW == total and nchunk * _CHUNK == per_w and d == _D
    assert nchunk % _NBUF == 0 and nchunk // _NBUF >= 2
    assert b % (2 * _CHUNK) == 0
    xt = x.T.reshape(_NW, nchunk, _CHUNK)
    tab_f32 = table.astype(jnp.float32)
    out = _build(b, h, nchunk)(xt, tab_f32)
    out_bf = jax.lax.bitcast_convert_type(out, jnp.bfloat16)
    out6 = out_bf.reshape(h, 8, b // _CHUNK, 4, _CHUNK, 2)
    return out6.transpose(2, 4, 0, 1, 3, 5).reshape(b, h, _D)
